# baseline (speedup 1.0000x reference)
import functools
import math

import jax
import jax.numpy as jnp
from jax import lax
from jax.experimental import pallas as pl
from jax.experimental.pallas import tpu as pltpu
from jax.experimental.pallas import tpu_sc as plsc

_EPS = 1e-5
_NC = 2
_NS = 16
_CH = 80


def _sc_gather_src_tgt(table, row, col, base_e, sub_e):
  n, f = table.shape
  nw = _NC * _NS
  per_w = sub_e // nw
  n_it = per_w // _CH
  st_a = (n // _NS) // 8 * 8
  st_last = n - 15 * st_a
  mesh = plsc.VectorSubcoreMesh(core_axis_name="c", subcore_axis_name="s")

  @functools.partial(
      pl.kernel,
      out_type=(jax.ShapeDtypeStruct((sub_e, f), jnp.float32),
                jax.ShapeDtypeStruct((sub_e, f), jnp.float32)),
      mesh=mesh,
      scratch_types=(
          [pltpu.VMEM((_CH,), jnp.int32)] * 4
          + [pltpu.VMEM((_CH, f), jnp.float32)] * 4
          + [pltpu.VMEM_SHARED((n, f), jnp.float32)]
          + [pltpu.SemaphoreType.DMA] * 12
      ),
  )
  def k(tab, row_h, col_h, src_h, tgt_h, ib0, ib1, ib2, ib3,
        bf0, bf1, bf2, bf3, shared,
        is0, is1, is2, is3, gs0, gs1, gs2, gs3, ws0, ws1, ws2, ws3):
    s = lax.axis_index("s")
    wid = s * _NC + lax.axis_index("c")
    base = base_e + wid * per_w
    obase = wid * per_w
    ibufs = [ib0, ib1, ib2, ib3]
    bufs = [bf0, bf1, bf2, bf3]
    isems = [is0, is1, is2, is3]
    gsems = [gs0, gs1, gs2, gs3]
    wsems = [ws0, ws1, ws2, ws3]

    @pl.when(s < _NS - 1)
    def _():
      pltpu.sync_copy(tab.at[pl.ds(s * st_a, st_a)],
                      shared.at[pl.ds(s * st_a, st_a)])

    @pl.when(s == _NS - 1)
    def _():
      pltpu.sync_copy(tab.at[pl.ds((_NS - 1) * st_a, st_last)],
                      shared.at[pl.ds((_NS - 1) * st_a, st_last)])

    plsc.subcore_barrier()

    nk = 2 * n_it
    srcs = [row_h, col_h]
    outs = [src_h, tgt_h]

    def body(it, carry):
      k0 = 4 * it
      for s4 in range(4):
        k = k0 + s4
        b = s4
        b2 = (s4 + 2) % 4
        b3 = (s4 + 1) % 4

        @pl.when(jnp.logical_and(k >= 4, k < nk + 4))
        def _():
          pltpu.make_async_copy(bufs[b], outs[s4 % 2].at[pl.ds(0, _CH)],
                                wsems[b]).wait()

        @pl.when(k < nk)
        def _():
          j = k // 2
          pltpu.async_copy(srcs[s4 % 2].at[pl.ds(base + j * _CH, _CH)],
                           ibufs[b], isems[b])

        @pl.when(jnp.logical_and(k >= 2, k < nk + 2))
        def _():
          pltpu.make_async_copy(srcs[0].at[pl.ds(0, _CH)], ibufs[b2],
                                isems[b2]).wait()
          pltpu.async_copy(shared.at[ibufs[b2]], bufs[b2], gsems[b2])

        @pl.when(jnp.logical_and(k >= 3, k < nk + 3))
        def _():
          j3 = jnp.maximum(k - 3, 0) // 2
          pltpu.make_async_copy(shared.at[pl.ds(0, _CH)], bufs[b3],
                                gsems[b3]).wait()
          pltpu.async_copy(bufs[b3],
                           outs[(s4 + 1) % 2].at[pl.ds(obase + j3 * _CH, _CH)],
                           wsems[b3])
      return carry

    lax.fori_loop(0, (nk + 8) // 4 + 1, body, 0)

  return k(table, row, col)


def _sc_scatter_add(msg, col, zeros, n_pad, base_e):
  sub_e, f = msg.shape
  nw = _NC * _NS
  per_w = sub_e // nw
  n_it = per_w // _CH
  stripe = n_pad // _NS
  n_z = stripe // _CH
  mesh = plsc.VectorSubcoreMesh(core_axis_name="c", subcore_axis_name="s")

  @functools.partial(
      pl.kernel,
      out_type=jax.ShapeDtypeStruct((2 * n_pad, f), jnp.float32),
      mesh=mesh,
      scratch_types=(
          [pltpu.VMEM((_CH,), jnp.int32)] * 4
          + [pltpu.VMEM((_CH, f), jnp.float32)] * 4
          + [pltpu.VMEM_SHARED((n_pad, f), jnp.float32)]
          + [pltpu.SemaphoreType.DMA] * 12
      ),
  )
  def k(msg_h, col_h, zero_h, out_h, ib0, ib1, ib2, ib3,
        bf0, bf1, bf2, bf3, shared,
        is0, is1, is2, is3, ls0, ls1, ls2, ls3, as0, as1, as2, as3):
    c = lax.axis_index("c")
    s = lax.axis_index("s")
    wid = s * _NC + c
    mbase = wid * per_w
    ibufs = [ib0, ib1, ib2, ib3]
    bufs = [bf0, bf1, bf2, bf3]
    isems = [is0, is1, is2, is3]
    lsems = [ls0, ls1, ls2, ls3]
    asems = [as0, as1, as2, as3]

    pltpu.sync_copy(zero_h, bf0)

    def zbody(z, carry):
      pltpu.sync_copy(bf0, shared.at[pl.ds(s * stripe + z * _CH, _CH)])
      return carry

    lax.fori_loop(0, n_z, zbody, 0)
    plsc.subcore_barrier()

    base = base_e + wid * per_w

    def body(it, carry):
      k0 = 4 * it
      for s4 in range(4):
        k = k0 + s4
        b = s4
        b2 = (s4 + 2) % 4

        @pl.when(jnp.logical_and(k >= 4, k < n_it + 4))
        def _():
          pltpu.make_async_copy(msg_h.at[pl.ds(0, _CH)], bufs[b],
                                asems[b]).wait()

        @pl.when(k < n_it)
        def _():
          pltpu.async_copy(col_h.at[pl.ds(base + k * _CH, _CH)], ibufs[b],
                           isems[b])
          pltpu.async_copy(msg_h.at[pl.ds(mbase + k * _CH, _CH)], bufs[b],
                           lsems[b])

        @pl.when(jnp.logical_and(k >= 2, k < n_it + 2))
        def _():
          pltpu.make_async_copy(col_h.at[pl.ds(0, _CH)], ibufs[b2],
                                isems[b2]).wait()
          pltpu.make_async_copy(msg_h.at[pl.ds(0, _CH)], bufs[b2],
                                lsems[b2]).wait()
          pltpu.async_copy(bufs[b2], shared.at[ibufs[b2]], asems[b2],
                           add=True)
      return carry

    lax.fori_loop(0, (n_it + 8) // 4 + 1, body, 0)
    plsc.subcore_barrier()

    def obody(z, carry):
      pltpu.sync_copy(shared.at[pl.ds(s * stripe + z * _CH, _CH)], bf0)
      pltpu.sync_copy(bf0, out_h.at[pl.ds(c * n_pad + s * stripe + z * _CH,
                                          _CH)])
      return carry

    lax.fori_loop(0, n_z, obody, 0)

  return k(msg, col, zeros)


_BE = 1280


def _stats_rows(v):
  s = jnp.sum(v, axis=0, keepdims=True)
  ss = jnp.sum(v * v, axis=0, keepdims=True)
  return jnp.concatenate(
      [s, ss, jnp.zeros((6, v.shape[1]), jnp.float32)], axis=0)


def _tc_atom0(x, atom_emb):
  n = x.shape[0]
  a, f = atom_emb.shape

  def body(x_ref, emb_ref, out_ref, st_ref):
    onehot = (x_ref[...][:, None]
              == lax.broadcasted_iota(jnp.int32, (1, a), 1)).astype(jnp.float32)
    v = jnp.dot(onehot, emb_ref[...], preferred_element_type=jnp.float32,
                precision=jax.lax.Precision.HIGHEST)
    out_ref[...] = v
    st_ref[...] = _stats_rows(v)

  return pl.pallas_call(
      body,
      out_shape=(jax.ShapeDtypeStruct((n, f), jnp.float32),
                 jax.ShapeDtypeStruct((8, f), jnp.float32)),
  )(x, atom_emb)


def _tc_bond0(attr3, bond_emb, e):
  b, f = bond_emb.shape
  grid = e // _BE

  def body(attr_ref, emb_ref, out_ref, st_ref):
    onehot = (attr_ref[0, 0, :][:, None]
              == lax.broadcasted_iota(jnp.int32, (1, b), 1)).astype(jnp.float32)
    v = jnp.dot(onehot, emb_ref[...], preferred_element_type=jnp.float32,
                precision=jax.lax.Precision.HIGHEST)
    out_ref[...] = v
    i = pl.program_id(0)

    @pl.when(i == 0)
    def _():
      st_ref[...] = jnp.zeros_like(st_ref)

    st_ref[...] += _stats_rows(v)

  return pl.pallas_call(
      body,
      grid=(grid,),
      in_specs=[pl.BlockSpec((1, 1, _BE), lambda i: (i, 0, 0)),
                pl.BlockSpec((b, f), lambda i: (0, 0))],
      out_specs=(pl.BlockSpec((_BE, f), lambda i: (i, 0)),
                 pl.BlockSpec((8, f), lambda i: (0, 0))),
      out_shape=(jax.ShapeDtypeStruct((e, f), jnp.float32),
                 jax.ShapeDtypeStruct((8, f), jnp.float32)),
  )(attr3, bond_emb)


def _tc_edge(src, tgt, bond, base_blk, a_sc, a_sh, b_sc, b_sh,
             w1a, w1b, w1c, b1, w2, b2, mw, mb):
  e, f = src.shape
  f2 = w1a.shape[1]
  grid = e // _BE

  def body(src_ref, tgt_ref, bond_ref, asc, ash, bsc, bsh,
           w1a_r, w1b_r, w1c_r, b1_r, w2_r, b2_r, mw_r, mb_r,
           bond_out, msg_out, st_ref):
    xs = src_ref[...] * asc[...] + ash[...]
    xt = tgt_ref[...] * asc[...] + ash[...]
    en = bond_ref[...] * bsc[...] + bsh[...]
    hcat = jnp.concatenate([xs, xt, en], axis=1)
    w1cat = jnp.concatenate([w1a_r[...], w1b_r[...], w1c_r[...]], axis=0)
    h = jnp.dot(hcat, w1cat, preferred_element_type=jnp.float32,
                precision=None) + b1_r[...]
    ne = jnp.dot(jnp.maximum(h, 0.0), w2_r[...],
                 preferred_element_type=jnp.float32, precision=None) + b2_r[...]
    bnew = bond_ref[...] + ne
    bond_out[...] = bnew
    msg_out[...] = (jnp.dot(xs, mw_r[...], preferred_element_type=jnp.float32, precision=None)
                    + mb_r[...]) * bnew
    i = pl.program_id(0)

    @pl.when(i == 0)
    def _():
      st_ref[...] = jnp.zeros_like(st_ref)

    st_ref[...] += _stats_rows(bnew)

  cst = lambda s: pl.BlockSpec(s, lambda i: tuple(0 for _ in s))
  blk = pl.BlockSpec((_BE, f), lambda i: (i, 0))
  bblk = pl.BlockSpec((_BE, f), lambda i: (base_blk + i, 0))
  return pl.pallas_call(
      body,
      grid=(grid,),
      in_specs=[blk, blk, bblk,
                cst((1, f)), cst((1, f)), cst((1, f)), cst((1, f)),
                cst((f, f2)), cst((f, f2)), cst((f, f2)), cst((1, f2)),
                cst((f2, f)), cst((1, f)), cst((f, f)), cst((1, f))],
      out_specs=(blk, blk, pl.BlockSpec((8, f), lambda i: (0, 0))),
      out_shape=(jax.ShapeDtypeStruct((e, f), jnp.float32),
                 jax.ShapeDtypeStruct((e, f), jnp.float32),
                 jax.ShapeDtypeStruct((8, f), jnp.float32)),
  )(src, tgt, bond, a_sc, a_sh, b_sc, b_sh, w1a, w1b, w1c, b1, w2, b2, mw, mb)


def _tc_node(aggr2a, aggr2b, atom, nw1, nb1, nw2, nb2, n_pad):
  n, f = atom.shape

  def body(a2a_ref, a2b_ref, atom_ref, w1_r, b1_r, w2_r, b2_r,
           out_ref, st_ref):
    a = (a2a_ref[0:n, :] + a2a_ref[n_pad:n_pad + n, :]
         + a2b_ref[0:n, :] + a2b_ref[n_pad:n_pad + n, :])
    h = jnp.maximum(
        jnp.dot(a, w1_r[...], preferred_element_type=jnp.float32, precision=None) + b1_r[...],
        0.0)
    upd = jnp.dot(h, w2_r[...], preferred_element_type=jnp.float32, precision=None) + b2_r[...]
    anew = atom_ref[...] + upd
    out_ref[...] = anew
    st_ref[...] = _stats_rows(anew)

  return pl.pallas_call(
      body,
      out_shape=(jax.ShapeDtypeStruct((n, f), jnp.float32),
                 jax.ShapeDtypeStruct((8, f), jnp.float32)),
  )(aggr2a, aggr2b, atom, nw1, nb1, nw2, nb2)


def _tc_final(bond, attr3, base_blk, out_w, out_b, mean_emb):
  e, f = bond.shape
  b = mean_emb.shape[0]
  grid = e // _BE

  def body(bond_ref, attr_ref, w_r, b_r, memb_r, out_ref):
    onehot = (attr_ref[0, 0, :][:, None]
              == lax.broadcasted_iota(jnp.int32, (1, b), 1)).astype(jnp.float32)
    mean = jnp.dot(onehot, memb_r[...], preferred_element_type=jnp.float32,
                   precision=jax.lax.Precision.HIGHEST)
    out_ref[...] = (jnp.dot(bond_ref[...], w_r[...],
                            preferred_element_type=jnp.float32, precision=None)
                    + b_r[...] + mean)

  cst = lambda s: pl.BlockSpec(s, lambda i: tuple(0 for _ in s))
  return pl.pallas_call(
      body,
      grid=(grid,),
      in_specs=[pl.BlockSpec((_BE, f), lambda i: (i, 0)),
                pl.BlockSpec((1, 1, _BE), lambda i: (base_blk + i, 0, 0)),
                cst((f, 1)), cst((1, 1)), cst((b, 1))],
      out_specs=pl.BlockSpec((_BE, 1), lambda i: (i, 0)),
      out_shape=jax.ShapeDtypeStruct((e, 1), jnp.float32),
  )(bond, attr3, out_w, out_b, mean_emb)


def _bn_coeffs(st, cnt, gamma, beta):
  m = st[0] / cnt
  var = st[1] / cnt - m * m
  scale = gamma / jnp.sqrt(var + _EPS)
  shift = beta - m * scale
  return scale.reshape(1, -1), shift.reshape(1, -1)


def kernel(x, edge_index, edge_attr, atom_emb, bond_emb, bond_mean_emb,
           bn_atom_gamma, bn_atom_beta, bn_bond_gamma, bn_bond_beta,
           edge_W1, edge_b1, edge_W2, edge_b2, msg_W, msg_b,
           node_W1, node_b1, node_W2, node_b2, out_W, out_b):
  n = x.shape[0]
  e = edge_attr.shape[0]
  f = atom_emb.shape[1]
  num_layers = edge_W1.shape[0]
  row = edge_index[0].astype(jnp.int32)
  col = edge_index[1].astype(jnp.int32)
  x = x.astype(jnp.int32)
  attr = edge_attr.astype(jnp.int32)
  attr3 = attr.reshape(e // _BE, 1, _BE)
  stripe = -(-(n // _NS) // _CH) * _CH
  n_pad = _NS * stripe
  zeros = jnp.zeros((_CH, f), jnp.float32)

  atom_state, ast = _tc_atom0(x, atom_emb)
  bond0, bst = _tc_bond0(attr3, bond_emb, e)

  nw_q = _NC * _NS * _CH
  quantum = nw_q * _BE // math.gcd(nw_q, _BE)
  nq = e // quantum
  sub1 = (nq // 2) * quantum
  halves = [(0, sub1), (sub1, e - sub1)]
  bonds = [None, None]
  bases = [0, sub1]

  for l in range(num_layers):
    a_sc, a_sh = _bn_coeffs(ast, n, bn_atom_gamma[l], bn_atom_beta[l])
    b_sc, b_sh = _bn_coeffs(bst, e, bn_bond_gamma[l], bn_bond_beta[l])
    w1 = edge_W1[l]
    gath = [_sc_gather_src_tgt(atom_state, row, col, b0, se)
            for (b0, se) in halves]
    sts = []
    msgs = []
    for h, (b0, se) in enumerate(halves):
      src_h, tgt_h = gath[h]
      bond_in = bond0 if l == 0 else bonds[h]
      bblk = (b0 // _BE) if l == 0 else 0
      bond_h, msg_h, st_h = _tc_edge(
          src_h, tgt_h, bond_in, bblk, a_sc, a_sh, b_sc, b_sh,
          w1[:f], w1[f:2 * f], w1[2 * f:], edge_b1[l].reshape(1, -1),
          edge_W2[l], edge_b2[l].reshape(1, -1),
          msg_W[l], msg_b[l].reshape(1, -1))
      bonds[h] = bond_h
      msgs.append(msg_h)
      sts.append(st_h)
    bst = sts[0] + sts[1]
    ag_a = _sc_scatter_add(msgs[0], col, zeros, n_pad, bases[0])
    ag_b = _sc_scatter_add(msgs[1], col, zeros, n_pad, bases[1])
    atom_state, ast = _tc_node(
        ag_a, ag_b, atom_state, node_W1[l], node_b1[l].reshape(1, -1),
        node_W2[l], node_b2[l].reshape(1, -1), n_pad)

  preds = [_tc_final(bonds[h], attr3, bases[h] // _BE, out_W,
                     out_b.reshape(1, 1), bond_mean_emb)
           for h in range(2)]
  return jnp.concatenate(preds, axis=0).reshape(e)


# speedup vs baseline: 1.1686x; 1.1686x over previous
import functools

import jax
import jax.numpy as jnp
from jax import lax
from jax.experimental import pallas as pl
from jax.experimental.pallas import tpu as pltpu
from jax.experimental.pallas import tpu_sc as plsc

_EPS = 1e-5
_NC = 2
_NS = 16
_CH = 80


def _sc_gather_src_tgt(table, row, col):
  n, f = table.shape
  e = row.shape[0]
  nw = _NC * _NS
  per_w = e // nw
  n_it = per_w // _CH
  st_a = (n // _NS) // 8 * 8
  st_last = n - 15 * st_a
  mesh = plsc.VectorSubcoreMesh(core_axis_name="c", subcore_axis_name="s")

  @functools.partial(
      pl.kernel,
      out_type=(jax.ShapeDtypeStruct((e, f), jnp.float32),
                jax.ShapeDtypeStruct((e, f), jnp.float32)),
      mesh=mesh,
      scratch_types=(
          [pltpu.VMEM((_CH,), jnp.int32)] * 4
          + [pltpu.VMEM((_CH, f), jnp.float32)] * 4
          + [pltpu.VMEM_SHARED((n, f), jnp.float32)]
          + [pltpu.SemaphoreType.DMA] * 12
      ),
  )
  def k(tab, row_h, col_h, src_h, tgt_h, ib0, ib1, ib2, ib3,
        bf0, bf1, bf2, bf3, shared,
        is0, is1, is2, is3, gs0, gs1, gs2, gs3, ws0, ws1, ws2, ws3):
    s = lax.axis_index("s")
    wid = s * _NC + lax.axis_index("c")
    base = wid * per_w
    ibufs = [ib0, ib1, ib2, ib3]
    bufs = [bf0, bf1, bf2, bf3]
    isems = [is0, is1, is2, is3]
    gsems = [gs0, gs1, gs2, gs3]
    wsems = [ws0, ws1, ws2, ws3]

    @pl.when(s < _NS - 1)
    def _():
      pltpu.sync_copy(tab.at[pl.ds(s * st_a, st_a)],
                      shared.at[pl.ds(s * st_a, st_a)])

    @pl.when(s == _NS - 1)
    def _():
      pltpu.sync_copy(tab.at[pl.ds((_NS - 1) * st_a, st_last)],
                      shared.at[pl.ds((_NS - 1) * st_a, st_last)])

    plsc.subcore_barrier()

    nk = 2 * n_it
    srcs = [row_h, col_h]
    outs = [src_h, tgt_h]

    def body(it, carry):
      k0 = 4 * it
      for s4 in range(4):
        k = k0 + s4
        b = s4
        b2 = (s4 + 2) % 4
        b3 = (s4 + 1) % 4

        @pl.when(jnp.logical_and(k >= 4, k < nk + 4))
        def _():
          pltpu.make_async_copy(bufs[b], outs[s4 % 2].at[pl.ds(0, _CH)],
                                wsems[b]).wait()

        @pl.when(k < nk)
        def _():
          j = k // 2
          pltpu.async_copy(srcs[s4 % 2].at[pl.ds(base + j * _CH, _CH)],
                           ibufs[b], isems[b])

        @pl.when(jnp.logical_and(k >= 2, k < nk + 2))
        def _():
          pltpu.make_async_copy(srcs[0].at[pl.ds(0, _CH)], ibufs[b2],
                                isems[b2]).wait()
          pltpu.async_copy(shared.at[ibufs[b2]], bufs[b2], gsems[b2])

        @pl.when(jnp.logical_and(k >= 3, k < nk + 3))
        def _():
          j3 = jnp.maximum(k - 3, 0) // 2
          pltpu.make_async_copy(shared.at[pl.ds(0, _CH)], bufs[b3],
                                gsems[b3]).wait()
          pltpu.async_copy(bufs[b3],
                           outs[(s4 + 1) % 2].at[pl.ds(base + j3 * _CH, _CH)],
                           wsems[b3])
      return carry

    lax.fori_loop(0, (nk + 8) // 4 + 1, body, 0)

  return k(table, row, col)


def _sc_scatter_add(msg, col, zeros, n_pad):
  e, f = msg.shape
  nw = _NC * _NS
  per_w = e // nw
  n_it = per_w // _CH
  stripe = n_pad // _NS
  n_z = stripe // _CH
  mesh = plsc.VectorSubcoreMesh(core_axis_name="c", subcore_axis_name="s")

  @functools.partial(
      pl.kernel,
      out_type=jax.ShapeDtypeStruct((2 * n_pad, f), jnp.float32),
      mesh=mesh,
      scratch_types=(
          [pltpu.VMEM((_CH,), jnp.int32)] * 4
          + [pltpu.VMEM((_CH, f), jnp.float32)] * 4
          + [pltpu.VMEM_SHARED((n_pad, f), jnp.float32)]
          + [pltpu.SemaphoreType.DMA] * 12
      ),
  )
  def k(msg_h, col_h, zero_h, out_h, ib0, ib1, ib2, ib3,
        bf0, bf1, bf2, bf3, shared,
        is0, is1, is2, is3, ls0, ls1, ls2, ls3, as0, as1, as2, as3):
    c = lax.axis_index("c")
    s = lax.axis_index("s")
    wid = s * _NC + c
    ibufs = [ib0, ib1, ib2, ib3]
    bufs = [bf0, bf1, bf2, bf3]
    isems = [is0, is1, is2, is3]
    lsems = [ls0, ls1, ls2, ls3]
    asems = [as0, as1, as2, as3]

    pltpu.sync_copy(zero_h, bf0)

    def zbody(z, carry):
      pltpu.sync_copy(bf0, shared.at[pl.ds(s * stripe + z * _CH, _CH)])
      return carry

    lax.fori_loop(0, n_z, zbody, 0)
    plsc.subcore_barrier()

    base = wid * per_w

    def body(it, carry):
      k0 = 4 * it
      for s4 in range(4):
        k = k0 + s4
        b = s4
        b2 = (s4 + 2) % 4

        @pl.when(jnp.logical_and(k >= 4, k < n_it + 4))
        def _():
          pltpu.make_async_copy(msg_h.at[pl.ds(0, _CH)], bufs[b],
                                asems[b]).wait()

        @pl.when(k < n_it)
        def _():
          o = base + k * _CH
          pltpu.async_copy(col_h.at[pl.ds(o, _CH)], ibufs[b], isems[b])
          pltpu.async_copy(msg_h.at[pl.ds(o, _CH)], bufs[b], lsems[b])

        @pl.when(jnp.logical_and(k >= 2, k < n_it + 2))
        def _():
          pltpu.make_async_copy(col_h.at[pl.ds(0, _CH)], ibufs[b2],
                                isems[b2]).wait()
          pltpu.make_async_copy(msg_h.at[pl.ds(0, _CH)], bufs[b2],
                                lsems[b2]).wait()
          pltpu.async_copy(bufs[b2], shared.at[ibufs[b2]], asems[b2],
                           add=True)
      return carry

    lax.fori_loop(0, (n_it + 8) // 4 + 1, body, 0)
    plsc.subcore_barrier()

    def obody(z, carry):
      pltpu.sync_copy(shared.at[pl.ds(s * stripe + z * _CH, _CH)], bf0)
      pltpu.sync_copy(bf0, out_h.at[pl.ds(c * n_pad + s * stripe + z * _CH,
                                          _CH)])
      return carry

    lax.fori_loop(0, n_z, obody, 0)

  return k(msg, col, zeros)


_BE = 3200


def _stats_rows(v):
  s = jnp.sum(v, axis=0, keepdims=True)
  ss = jnp.sum(v * v, axis=0, keepdims=True)
  return jnp.concatenate(
      [s, ss, jnp.zeros((6, v.shape[1]), jnp.float32)], axis=0)


def _tc_atom0(x, atom_emb):
  n = x.shape[0]
  a, f = atom_emb.shape

  def body(x_ref, emb_ref, out_ref, st_ref):
    onehot = (x_ref[...][:, None]
              == lax.broadcasted_iota(jnp.int32, (1, a), 1)).astype(jnp.float32)
    v = jnp.dot(onehot, emb_ref[...], preferred_element_type=jnp.float32,
                precision=jax.lax.Precision.HIGHEST)
    out_ref[...] = v
    st_ref[...] = _stats_rows(v)

  return pl.pallas_call(
      body,
      out_shape=(jax.ShapeDtypeStruct((n, f), jnp.float32),
                 jax.ShapeDtypeStruct((8, f), jnp.float32)),
  )(x, atom_emb)


def _tc_bond0(attr3, bond_emb, e):
  b, f = bond_emb.shape
  grid = e // _BE

  def body(attr_ref, emb_ref, out_ref, st_ref):
    onehot = (attr_ref[0, 0, :][:, None]
              == lax.broadcasted_iota(jnp.int32, (1, b), 1)).astype(jnp.float32)
    v = jnp.dot(onehot, emb_ref[...], preferred_element_type=jnp.float32,
                precision=jax.lax.Precision.HIGHEST)
    out_ref[...] = v
    i = pl.program_id(0)

    @pl.when(i == 0)
    def _():
      st_ref[...] = jnp.zeros_like(st_ref)

    st_ref[...] += _stats_rows(v)

  return pl.pallas_call(
      body,
      grid=(grid,),
      in_specs=[pl.BlockSpec((1, 1, _BE), lambda i: (i, 0, 0)),
                pl.BlockSpec((b, f), lambda i: (0, 0))],
      out_specs=(pl.BlockSpec((_BE, f), lambda i: (i, 0)),
                 pl.BlockSpec((8, f), lambda i: (0, 0))),
      out_shape=(jax.ShapeDtypeStruct((e, f), jnp.float32),
                 jax.ShapeDtypeStruct((8, f), jnp.float32)),
  )(attr3, bond_emb)


def _tc_edge(src, tgt, bond, a_sc, a_sh, b_sc, b_sh,
             w1a, w1b, w1c, b1, w2, b2, mw, mb):
  e, f = bond.shape
  f2 = w1a.shape[1]
  grid = e // _BE

  def body(src_ref, tgt_ref, bond_ref, asc, ash, bsc, bsh,
           w1a_r, w1b_r, w1c_r, b1_r, w2_r, b2_r, mw_r, mb_r,
           bond_out, msg_out, st_ref):
    xs = src_ref[...] * asc[...] + ash[...]
    xt = tgt_ref[...] * asc[...] + ash[...]
    en = bond_ref[...] * bsc[...] + bsh[...]
    hcat = jnp.concatenate([xs, xt, en], axis=1)
    w1cat = jnp.concatenate([w1a_r[...], w1b_r[...], w1c_r[...]], axis=0)
    h = jnp.dot(hcat, w1cat, preferred_element_type=jnp.float32,
                precision=None) + b1_r[...]
    ne = jnp.dot(jnp.maximum(h, 0.0), w2_r[...],
                 preferred_element_type=jnp.float32, precision=None) + b2_r[...]
    bnew = bond_ref[...] + ne
    bond_out[...] = bnew
    msg_out[...] = (jnp.dot(xs, mw_r[...], preferred_element_type=jnp.float32, precision=None)
                    + mb_r[...]) * bnew
    i = pl.program_id(0)

    @pl.when(i == 0)
    def _():
      st_ref[...] = jnp.zeros_like(st_ref)

    st_ref[...] += _stats_rows(bnew)

  cst = lambda s: pl.BlockSpec(s, lambda i: tuple(0 for _ in s))
  blk = pl.BlockSpec((_BE, f), lambda i: (i, 0))
  return pl.pallas_call(
      body,
      grid=(grid,),
      in_specs=[blk, blk, blk,
                cst((1, f)), cst((1, f)), cst((1, f)), cst((1, f)),
                cst((f, f2)), cst((f, f2)), cst((f, f2)), cst((1, f2)),
                cst((f2, f)), cst((1, f)), cst((f, f)), cst((1, f))],
      out_specs=(blk, blk, pl.BlockSpec((8, f), lambda i: (0, 0))),
      out_shape=(jax.ShapeDtypeStruct((e, f), jnp.float32),
                 jax.ShapeDtypeStruct((e, f), jnp.float32),
                 jax.ShapeDtypeStruct((8, f), jnp.float32)),
  )(src, tgt, bond, a_sc, a_sh, b_sc, b_sh, w1a, w1b, w1c, b1, w2, b2, mw, mb)


def _tc_node(aggr2, atom, nw1, nb1, nw2, nb2, n_pad):
  n, f = atom.shape

  def body(a2_ref, atom_ref, w1_r, b1_r, w2_r, b2_r, out_ref, st_ref):
    a = a2_ref[0:n, :] + a2_ref[n_pad:n_pad + n, :]
    h = jnp.maximum(
        jnp.dot(a, w1_r[...], preferred_element_type=jnp.float32, precision=None) + b1_r[...],
        0.0)
    upd = jnp.dot(h, w2_r[...], preferred_element_type=jnp.float32, precision=None) + b2_r[...]
    anew = atom_ref[...] + upd
    out_ref[...] = anew
    st_ref[...] = _stats_rows(anew)

  return pl.pallas_call(
      body,
      out_shape=(jax.ShapeDtypeStruct((n, f), jnp.float32),
                 jax.ShapeDtypeStruct((8, f), jnp.float32)),
  )(aggr2, atom, nw1, nb1, nw2, nb2)


def _tc_final(bond, attr3, out_w, out_b, mean_emb):
  e, f = bond.shape
  b = mean_emb.shape[0]
  grid = e // _BE

  def body(bond_ref, attr_ref, w_r, b_r, memb_r, out_ref):
    onehot = (attr_ref[0, 0, :][:, None]
              == lax.broadcasted_iota(jnp.int32, (1, b), 1)).astype(jnp.float32)
    mean = jnp.dot(onehot, memb_r[...], preferred_element_type=jnp.float32,
                   precision=jax.lax.Precision.HIGHEST)
    out_ref[...] = (jnp.dot(bond_ref[...], w_r[...],
                            preferred_element_type=jnp.float32, precision=None)
                    + b_r[...] + mean)

  cst = lambda s: pl.BlockSpec(s, lambda i: tuple(0 for _ in s))
  return pl.pallas_call(
      body,
      grid=(grid,),
      in_specs=[pl.BlockSpec((_BE, f), lambda i: (i, 0)),
                pl.BlockSpec((1, 1, _BE), lambda i: (i, 0, 0)),
                cst((f, 1)), cst((1, 1)), cst((b, 1))],
      out_specs=pl.BlockSpec((_BE, 1), lambda i: (i, 0)),
      out_shape=jax.ShapeDtypeStruct((e, 1), jnp.float32),
  )(bond, attr3, out_w, out_b, mean_emb)


def _bn_coeffs(st, cnt, gamma, beta):
  m = st[0] / cnt
  var = st[1] / cnt - m * m
  scale = gamma / jnp.sqrt(var + _EPS)
  shift = beta - m * scale
  return scale.reshape(1, -1), shift.reshape(1, -1)


def kernel(x, edge_index, edge_attr, atom_emb, bond_emb, bond_mean_emb,
           bn_atom_gamma, bn_atom_beta, bn_bond_gamma, bn_bond_beta,
           edge_W1, edge_b1, edge_W2, edge_b2, msg_W, msg_b,
           node_W1, node_b1, node_W2, node_b2, out_W, out_b):
  n = x.shape[0]
  e = edge_attr.shape[0]
  f = atom_emb.shape[1]
  num_layers = edge_W1.shape[0]
  row = edge_index[0].astype(jnp.int32)
  col = edge_index[1].astype(jnp.int32)
  x = x.astype(jnp.int32)
  attr = edge_attr.astype(jnp.int32)
  attr3 = attr.reshape(e // _BE, 1, _BE)
  stripe = -(-(n // _NS) // _CH) * _CH
  n_pad = _NS * stripe
  zeros = jnp.zeros((_CH, f), jnp.float32)

  atom_state, ast = _tc_atom0(x, atom_emb)
  bond_state, bst = _tc_bond0(attr3, bond_emb, e)

  for l in range(num_layers):
    a_sc, a_sh = _bn_coeffs(ast, n, bn_atom_gamma[l], bn_atom_beta[l])
    b_sc, b_sh = _bn_coeffs(bst, e, bn_bond_gamma[l], bn_bond_beta[l])
    src, tgt = _sc_gather_src_tgt(atom_state, row, col)
    w1 = edge_W1[l]
    bond_state, msg, bst = _tc_edge(
        src, tgt, bond_state, a_sc, a_sh, b_sc, b_sh,
        w1[:f], w1[f:2 * f], w1[2 * f:], edge_b1[l].reshape(1, -1),
        edge_W2[l], edge_b2[l].reshape(1, -1),
        msg_W[l], msg_b[l].reshape(1, -1))
    aggr2 = _sc_scatter_add(msg, col, zeros, n_pad)
    atom_state, ast = _tc_node(
        aggr2, atom_state, node_W1[l], node_b1[l].reshape(1, -1),
        node_W2[l], node_b2[l].reshape(1, -1), n_pad)

  pred = _tc_final(bond_state, attr3, out_W, out_b.reshape(1, 1),
                   bond_mean_emb)
  return pred.reshape(e)


# speedup vs baseline: 1.2523x; 1.0717x over previous
import functools

import jax
import jax.numpy as jnp
from jax import lax
from jax.experimental import pallas as pl
from jax.experimental.pallas import tpu as pltpu
from jax.experimental.pallas import tpu_sc as plsc

_EPS = 1e-5
_NC = 2
_NS = 16
_CH = 80


def _sc_gather_src_tgt(table, row, col):
  n, f = table.shape
  e = row.shape[0]
  nw = _NC * _NS
  per_w = e // nw
  n_it = per_w // _CH
  st_a = (n // _NS) // 8 * 8
  st_last = n - 15 * st_a
  mesh = plsc.VectorSubcoreMesh(core_axis_name="c", subcore_axis_name="s")

  @functools.partial(
      pl.kernel,
      out_type=(jax.ShapeDtypeStruct((e, f), jnp.float32),
                jax.ShapeDtypeStruct((e, f), jnp.float32)),
      mesh=mesh,
      scratch_types=(
          [pltpu.VMEM((_CH,), jnp.int32)] * 4
          + [pltpu.VMEM((_CH, f), jnp.float32)] * 4
          + [pltpu.VMEM_SHARED((n, f), jnp.float32)]
          + [pltpu.SemaphoreType.DMA] * 12
      ),
  )
  def k(tab, row_h, col_h, src_h, tgt_h, ib0, ib1, ib2, ib3,
        bf0, bf1, bf2, bf3, shared,
        is0, is1, is2, is3, gs0, gs1, gs2, gs3, ws0, ws1, ws2, ws3):
    s = lax.axis_index("s")
    wid = s * _NC + lax.axis_index("c")
    base = wid * per_w
    ibufs = [ib0, ib1, ib2, ib3]
    bufs = [bf0, bf1, bf2, bf3]
    isems = [is0, is1, is2, is3]
    gsems = [gs0, gs1, gs2, gs3]
    wsems = [ws0, ws1, ws2, ws3]

    @pl.when(s < _NS - 1)
    def _():
      pltpu.sync_copy(tab.at[pl.ds(s * st_a, st_a)],
                      shared.at[pl.ds(s * st_a, st_a)])

    @pl.when(s == _NS - 1)
    def _():
      pltpu.sync_copy(tab.at[pl.ds((_NS - 1) * st_a, st_last)],
                      shared.at[pl.ds((_NS - 1) * st_a, st_last)])

    plsc.subcore_barrier()

    nk = 2 * n_it
    srcs = [row_h, col_h]
    outs = [src_h, tgt_h]

    def body(it, carry):
      k0 = 4 * it
      for s4 in range(4):
        k = k0 + s4
        b = s4
        b2 = (s4 + 2) % 4
        b3 = (s4 + 1) % 4

        @pl.when(jnp.logical_and(k >= 4, k < nk + 4))
        def _():
          pltpu.make_async_copy(bufs[b], outs[s4 % 2].at[pl.ds(0, _CH)],
                                wsems[b]).wait()

        @pl.when(k < nk)
        def _():
          j = k // 2
          pltpu.async_copy(srcs[s4 % 2].at[pl.ds(base + j * _CH, _CH)],
                           ibufs[b], isems[b])

        @pl.when(jnp.logical_and(k >= 2, k < nk + 2))
        def _():
          pltpu.make_async_copy(srcs[0].at[pl.ds(0, _CH)], ibufs[b2],
                                isems[b2]).wait()
          pltpu.async_copy(shared.at[ibufs[b2]], bufs[b2], gsems[b2])

        @pl.when(jnp.logical_and(k >= 3, k < nk + 3))
        def _():
          j3 = jnp.maximum(k - 3, 0) // 2
          pltpu.make_async_copy(shared.at[pl.ds(0, _CH)], bufs[b3],
                                gsems[b3]).wait()
          pltpu.async_copy(bufs[b3],
                           outs[(s4 + 1) % 2].at[pl.ds(base + j3 * _CH, _CH)],
                           wsems[b3])
      return carry

    lax.fori_loop(0, (nk + 8) // 4 + 1, body, 0)

  return k(table, row, col)


def _sc_scatter_add(msg, col, zeros, n_pad):
  e, f = msg.shape
  nw = _NC * _NS
  per_w = e // nw
  n_it = per_w // _CH
  stripe = n_pad // _NS
  n_z = stripe // _CH
  mesh = plsc.VectorSubcoreMesh(core_axis_name="c", subcore_axis_name="s")

  @functools.partial(
      pl.kernel,
      out_type=jax.ShapeDtypeStruct((2 * n_pad, f), jnp.float32),
      mesh=mesh,
      scratch_types=(
          [pltpu.VMEM((_CH,), jnp.int32)] * 4
          + [pltpu.VMEM((_CH, f), jnp.float32)] * 4
          + [pltpu.VMEM_SHARED((n_pad, f), jnp.float32)]
          + [pltpu.SemaphoreType.DMA] * 12
      ),
  )
  def k(msg_h, col_h, zero_h, out_h, ib0, ib1, ib2, ib3,
        bf0, bf1, bf2, bf3, shared,
        is0, is1, is2, is3, ls0, ls1, ls2, ls3, as0, as1, as2, as3):
    c = lax.axis_index("c")
    s = lax.axis_index("s")
    wid = s * _NC + c
    ibufs = [ib0, ib1, ib2, ib3]
    bufs = [bf0, bf1, bf2, bf3]
    isems = [is0, is1, is2, is3]
    lsems = [ls0, ls1, ls2, ls3]
    asems = [as0, as1, as2, as3]

    pltpu.sync_copy(zero_h, bf0)

    def zbody(z, carry):
      pltpu.sync_copy(bf0, shared.at[pl.ds(s * stripe + z * _CH, _CH)])
      return carry

    lax.fori_loop(0, n_z, zbody, 0)
    plsc.subcore_barrier()

    base = wid * per_w

    def body(it, carry):
      k0 = 4 * it
      for s4 in range(4):
        k = k0 + s4
        b = s4
        b2 = (s4 + 2) % 4

        @pl.when(jnp.logical_and(k >= 4, k < n_it + 4))
        def _():
          pltpu.make_async_copy(msg_h.at[pl.ds(0, _CH)], bufs[b],
                                asems[b]).wait()

        @pl.when(k < n_it)
        def _():
          o = base + k * _CH
          pltpu.async_copy(col_h.at[pl.ds(o, _CH)], ibufs[b], isems[b])
          pltpu.async_copy(msg_h.at[pl.ds(o, _CH)], bufs[b], lsems[b])

        @pl.when(jnp.logical_and(k >= 2, k < n_it + 2))
        def _():
          pltpu.make_async_copy(col_h.at[pl.ds(0, _CH)], ibufs[b2],
                                isems[b2]).wait()
          pltpu.make_async_copy(msg_h.at[pl.ds(0, _CH)], bufs[b2],
                                lsems[b2]).wait()
          pltpu.async_copy(bufs[b2], shared.at[ibufs[b2]], asems[b2],
                           add=True)
      return carry

    lax.fori_loop(0, (n_it + 8) // 4 + 1, body, 0)
    plsc.subcore_barrier()

    def obody(z, carry):
      pltpu.sync_copy(shared.at[pl.ds(s * stripe + z * _CH, _CH)], bf0)
      pltpu.sync_copy(bf0, out_h.at[pl.ds(c * n_pad + s * stripe + z * _CH,
                                          _CH)])
      return carry

    lax.fori_loop(0, n_z, obody, 0)

  return k(msg, col, zeros)


_BE = 6400


def _stats_rows(v):
  s = jnp.sum(v, axis=0, keepdims=True)
  ss = jnp.sum(v * v, axis=0, keepdims=True)
  return jnp.concatenate(
      [s, ss, jnp.zeros((6, v.shape[1]), jnp.float32)], axis=0)


def _tc_atom0(x, atom_emb):
  n = x.shape[0]
  a, f = atom_emb.shape

  def body(x_ref, emb_ref, out_ref, st_ref):
    onehot = (x_ref[...][:, None]
              == lax.broadcasted_iota(jnp.int32, (1, a), 1)).astype(jnp.float32)
    v = jnp.dot(onehot, emb_ref[...], preferred_element_type=jnp.float32,
                precision=jax.lax.Precision.HIGHEST)
    out_ref[...] = v
    st_ref[...] = _stats_rows(v)

  return pl.pallas_call(
      body,
      out_shape=(jax.ShapeDtypeStruct((n, f), jnp.float32),
                 jax.ShapeDtypeStruct((8, f), jnp.float32)),
  )(x, atom_emb)


def _tc_bond0(attr3, bond_emb, e):
  b, f = bond_emb.shape
  grid = e // _BE

  def body(attr_ref, emb_ref, out_ref, st_ref):
    onehot = (attr_ref[0, 0, :][:, None]
              == lax.broadcasted_iota(jnp.int32, (1, b), 1)).astype(jnp.float32)
    v = jnp.dot(onehot, emb_ref[...], preferred_element_type=jnp.float32,
                precision=jax.lax.Precision.HIGHEST)
    out_ref[...] = v
    i = pl.program_id(0)

    @pl.when(i == 0)
    def _():
      st_ref[...] = jnp.zeros_like(st_ref)

    st_ref[...] += _stats_rows(v)

  return pl.pallas_call(
      body,
      grid=(grid,),
      in_specs=[pl.BlockSpec((1, 1, _BE), lambda i: (i, 0, 0)),
                pl.BlockSpec((b, f), lambda i: (0, 0))],
      out_specs=(pl.BlockSpec((_BE, f), lambda i: (i, 0)),
                 pl.BlockSpec((8, f), lambda i: (0, 0))),
      out_shape=(jax.ShapeDtypeStruct((e, f), jnp.float32),
                 jax.ShapeDtypeStruct((8, f), jnp.float32)),
  )(attr3, bond_emb)


def _tc_edge(src, tgt, bond, a_sc, a_sh, b_sc, b_sh,
             w1a, w1b, w1c, b1, w2, b2, mw, mb):
  e, f = bond.shape
  f2 = w1a.shape[1]
  grid = e // _BE

  def body(src_ref, tgt_ref, bond_ref, asc, ash, bsc, bsh,
           w1a_r, w1b_r, w1c_r, b1_r, w2_r, b2_r, mw_r, mb_r,
           bond_out, msg_out, st_ref):
    xs = src_ref[...] * asc[...] + ash[...]
    xt = tgt_ref[...] * asc[...] + ash[...]
    en = bond_ref[...] * bsc[...] + bsh[...]
    hcat = jnp.concatenate([xs, xt, en], axis=1)
    w1cat = jnp.concatenate([w1a_r[...], w1b_r[...], w1c_r[...]], axis=0)
    h = jnp.dot(hcat, w1cat, preferred_element_type=jnp.float32,
                precision=None) + b1_r[...]
    ne = jnp.dot(jnp.maximum(h, 0.0), w2_r[...],
                 preferred_element_type=jnp.float32, precision=None) + b2_r[...]
    bnew = bond_ref[...] + ne
    bond_out[...] = bnew
    msg_out[...] = (jnp.dot(xs, mw_r[...], preferred_element_type=jnp.float32, precision=None)
                    + mb_r[...]) * bnew
    i = pl.program_id(0)

    @pl.when(i == 0)
    def _():
      st_ref[...] = jnp.zeros_like(st_ref)

    st_ref[...] += _stats_rows(bnew)

  cst = lambda s: pl.BlockSpec(s, lambda i: tuple(0 for _ in s))
  blk = pl.BlockSpec((_BE, f), lambda i: (i, 0))
  return pl.pallas_call(
      body,
      grid=(grid,),
      in_specs=[blk, blk, blk,
                cst((1, f)), cst((1, f)), cst((1, f)), cst((1, f)),
                cst((f, f2)), cst((f, f2)), cst((f, f2)), cst((1, f2)),
                cst((f2, f)), cst((1, f)), cst((f, f)), cst((1, f))],
      out_specs=(blk, blk, pl.BlockSpec((8, f), lambda i: (0, 0))),
      out_shape=(jax.ShapeDtypeStruct((e, f), jnp.float32),
                 jax.ShapeDtypeStruct((e, f), jnp.float32),
                 jax.ShapeDtypeStruct((8, f), jnp.float32)),
  )(src, tgt, bond, a_sc, a_sh, b_sc, b_sh, w1a, w1b, w1c, b1, w2, b2, mw, mb)


def _tc_node(aggr2, atom, nw1, nb1, nw2, nb2, n_pad):
  n, f = atom.shape

  def body(a2_ref, atom_ref, w1_r, b1_r, w2_r, b2_r, out_ref, st_ref):
    a = a2_ref[0:n, :] + a2_ref[n_pad:n_pad + n, :]
    h = jnp.maximum(
        jnp.dot(a, w1_r[...], preferred_element_type=jnp.float32, precision=None) + b1_r[...],
        0.0)
    upd = jnp.dot(h, w2_r[...], preferred_element_type=jnp.float32, precision=None) + b2_r[...]
    anew = atom_ref[...] + upd
    out_ref[...] = anew
    st_ref[...] = _stats_rows(anew)

  return pl.pallas_call(
      body,
      out_shape=(jax.ShapeDtypeStruct((n, f), jnp.float32),
                 jax.ShapeDtypeStruct((8, f), jnp.float32)),
  )(aggr2, atom, nw1, nb1, nw2, nb2)


def _tc_final(bond, attr3, out_w, out_b, mean_emb):
  e, f = bond.shape
  b = mean_emb.shape[0]
  grid = e // _BE

  def body(bond_ref, attr_ref, w_r, b_r, memb_r, out_ref):
    onehot = (attr_ref[0, 0, :][:, None]
              == lax.broadcasted_iota(jnp.int32, (1, b), 1)).astype(jnp.float32)
    mean = jnp.dot(onehot, memb_r[...], preferred_element_type=jnp.float32,
                   precision=jax.lax.Precision.HIGHEST)
    out_ref[...] = (jnp.dot(bond_ref[...], w_r[...],
                            preferred_element_type=jnp.float32, precision=None)
                    + b_r[...] + mean)

  cst = lambda s: pl.BlockSpec(s, lambda i: tuple(0 for _ in s))
  return pl.pallas_call(
      body,
      grid=(grid,),
      in_specs=[pl.BlockSpec((_BE, f), lambda i: (i, 0)),
                pl.BlockSpec((1, 1, _BE), lambda i: (i, 0, 0)),
                cst((f, 1)), cst((1, 1)), cst((b, 1))],
      out_specs=pl.BlockSpec((_BE, 1), lambda i: (i, 0)),
      out_shape=jax.ShapeDtypeStruct((e, 1), jnp.float32),
  )(bond, attr3, out_w, out_b, mean_emb)


def _bn_coeffs(st, cnt, gamma, beta):
  m = st[0] / cnt
  var = st[1] / cnt - m * m
  scale = gamma / jnp.sqrt(var + _EPS)
  shift = beta - m * scale
  return scale.reshape(1, -1), shift.reshape(1, -1)


def kernel(x, edge_index, edge_attr, atom_emb, bond_emb, bond_mean_emb,
           bn_atom_gamma, bn_atom_beta, bn_bond_gamma, bn_bond_beta,
           edge_W1, edge_b1, edge_W2, edge_b2, msg_W, msg_b,
           node_W1, node_b1, node_W2, node_b2, out_W, out_b):
  n = x.shape[0]
  e = edge_attr.shape[0]
  f = atom_emb.shape[1]
  num_layers = edge_W1.shape[0]
  row = edge_index[0].astype(jnp.int32)
  col = edge_index[1].astype(jnp.int32)
  x = x.astype(jnp.int32)
  attr = edge_attr.astype(jnp.int32)
  attr3 = attr.reshape(e // _BE, 1, _BE)
  stripe = -(-(n // _NS) // _CH) * _CH
  n_pad = _NS * stripe
  zeros = jnp.zeros((_CH, f), jnp.float32)

  atom_state, ast = _tc_atom0(x, atom_emb)
  bond_state, bst = _tc_bond0(attr3, bond_emb, e)

  for l in range(num_layers):
    a_sc, a_sh = _bn_coeffs(ast, n, bn_atom_gamma[l], bn_atom_beta[l])
    b_sc, b_sh = _bn_coeffs(bst, e, bn_bond_gamma[l], bn_bond_beta[l])
    src, tgt = _sc_gather_src_tgt(atom_state, row, col)
    w1 = edge_W1[l]
    bond_state, msg, bst = _tc_edge(
        src, tgt, bond_state, a_sc, a_sh, b_sc, b_sh,
        w1[:f], w1[f:2 * f], w1[2 * f:], edge_b1[l].reshape(1, -1),
        edge_W2[l], edge_b2[l].reshape(1, -1),
        msg_W[l], msg_b[l].reshape(1, -1))
    aggr2 = _sc_scatter_add(msg, col, zeros, n_pad)
    atom_state, ast = _tc_node(
        aggr2, atom_state, node_W1[l], node_b1[l].reshape(1, -1),
        node_W2[l], node_b2[l].reshape(1, -1), n_pad)

  pred = _tc_final(bond_state, attr3, out_W, out_b.reshape(1, 1),
                   bond_mean_emb)
  return pred.reshape(e)


# speedup vs baseline: 1.2664x; 1.0112x over previous
import functools

import jax
import jax.numpy as jnp
from jax import lax
from jax.experimental import pallas as pl
from jax.experimental.pallas import tpu as pltpu
from jax.experimental.pallas import tpu_sc as plsc

_EPS = 1e-5
_NC = 2
_NS = 16
_CH = 80


def _sc_gather_src_tgt(table, row, col):
  n, f = table.shape
  e = row.shape[0]
  nw = _NC * _NS
  per_w = e // nw
  n_it = per_w // _CH
  st_a = (n // _NS) // 8 * 8
  st_last = n - 15 * st_a
  mesh = plsc.VectorSubcoreMesh(core_axis_name="c", subcore_axis_name="s")

  @functools.partial(
      pl.kernel,
      out_type=(jax.ShapeDtypeStruct((e, f), jnp.float32),
                jax.ShapeDtypeStruct((e, f), jnp.float32)),
      mesh=mesh,
      scratch_types=(
          [pltpu.VMEM((_CH,), jnp.int32)] * 4
          + [pltpu.VMEM((_CH, f), jnp.float32)] * 4
          + [pltpu.VMEM_SHARED((n, f), jnp.float32)]
          + [pltpu.SemaphoreType.DMA] * 12
      ),
  )
  def k(tab, row_h, col_h, src_h, tgt_h, ib0, ib1, ib2, ib3,
        bf0, bf1, bf2, bf3, shared,
        is0, is1, is2, is3, gs0, gs1, gs2, gs3, ws0, ws1, ws2, ws3):
    s = lax.axis_index("s")
    wid = s * _NC + lax.axis_index("c")
    base = wid * per_w
    ibufs = [ib0, ib1, ib2, ib3]
    bufs = [bf0, bf1, bf2, bf3]
    isems = [is0, is1, is2, is3]
    gsems = [gs0, gs1, gs2, gs3]
    wsems = [ws0, ws1, ws2, ws3]

    @pl.when(s < _NS - 1)
    def _():
      pltpu.sync_copy(tab.at[pl.ds(s * st_a, st_a)],
                      shared.at[pl.ds(s * st_a, st_a)])

    @pl.when(s == _NS - 1)
    def _():
      pltpu.sync_copy(tab.at[pl.ds((_NS - 1) * st_a, st_last)],
                      shared.at[pl.ds((_NS - 1) * st_a, st_last)])

    plsc.subcore_barrier()

    nk = 2 * n_it
    srcs = [row_h, col_h]
    outs = [src_h, tgt_h]

    def body(it, carry):
      k0 = 4 * it
      for s4 in range(4):
        k = k0 + s4
        b = s4
        b2 = (s4 + 2) % 4
        b3 = (s4 + 1) % 4

        @pl.when(jnp.logical_and(k >= 4, k < nk + 4))
        def _():
          pltpu.make_async_copy(bufs[b], outs[s4 % 2].at[pl.ds(0, _CH)],
                                wsems[b]).wait()

        @pl.when(k < nk)
        def _():
          j = k // 2
          pltpu.async_copy(srcs[s4 % 2].at[pl.ds(base + j * _CH, _CH)],
                           ibufs[b], isems[b])

        @pl.when(jnp.logical_and(k >= 2, k < nk + 2))
        def _():
          pltpu.make_async_copy(srcs[0].at[pl.ds(0, _CH)], ibufs[b2],
                                isems[b2]).wait()
          pltpu.async_copy(shared.at[ibufs[b2]], bufs[b2], gsems[b2])

        @pl.when(jnp.logical_and(k >= 3, k < nk + 3))
        def _():
          j3 = jnp.maximum(k - 3, 0) // 2
          pltpu.make_async_copy(shared.at[pl.ds(0, _CH)], bufs[b3],
                                gsems[b3]).wait()
          pltpu.async_copy(bufs[b3],
                           outs[(s4 + 1) % 2].at[pl.ds(base + j3 * _CH, _CH)],
                           wsems[b3])
      return carry

    lax.fori_loop(0, (nk + 8) // 4 + 1, body, 0)

  return k(table, row, col)


def _sc_scatter_add(msg, col, zeros, n_pad):
  e, f = msg.shape
  nw = _NC * _NS
  per_w = e // nw
  n_it = per_w // _CH
  stripe = n_pad // _NS
  n_z = stripe // _CH
  mesh = plsc.VectorSubcoreMesh(core_axis_name="c", subcore_axis_name="s")

  @functools.partial(
      pl.kernel,
      out_type=jax.ShapeDtypeStruct((2 * n_pad, f), jnp.float32),
      mesh=mesh,
      scratch_types=(
          [pltpu.VMEM((_CH,), jnp.int32)] * 4
          + [pltpu.VMEM((_CH, f), jnp.float32)] * 4
          + [pltpu.VMEM_SHARED((n_pad, f), jnp.float32)]
          + [pltpu.SemaphoreType.DMA] * 12
      ),
  )
  def k(msg_h, col_h, zero_h, out_h, ib0, ib1, ib2, ib3,
        bf0, bf1, bf2, bf3, shared,
        is0, is1, is2, is3, ls0, ls1, ls2, ls3, as0, as1, as2, as3):
    c = lax.axis_index("c")
    s = lax.axis_index("s")
    wid = s * _NC + c
    ibufs = [ib0, ib1, ib2, ib3]
    bufs = [bf0, bf1, bf2, bf3]
    isems = [is0, is1, is2, is3]
    lsems = [ls0, ls1, ls2, ls3]
    asems = [as0, as1, as2, as3]

    pltpu.sync_copy(zero_h, bf0)

    def zbody(z, carry):
      pltpu.sync_copy(bf0, shared.at[pl.ds(s * stripe + z * _CH, _CH)])
      return carry

    lax.fori_loop(0, n_z, zbody, 0)
    plsc.subcore_barrier()

    base = wid * per_w

    def body(it, carry):
      k0 = 4 * it
      for s4 in range(4):
        k = k0 + s4
        b = s4
        b2 = (s4 + 2) % 4

        @pl.when(jnp.logical_and(k >= 4, k < n_it + 4))
        def _():
          pltpu.make_async_copy(msg_h.at[pl.ds(0, _CH)], bufs[b],
                                asems[b]).wait()

        @pl.when(k < n_it)
        def _():
          o = base + k * _CH
          pltpu.async_copy(col_h.at[pl.ds(o, _CH)], ibufs[b], isems[b])
          pltpu.async_copy(msg_h.at[pl.ds(o, _CH)], bufs[b], lsems[b])

        @pl.when(jnp.logical_and(k >= 2, k < n_it + 2))
        def _():
          pltpu.make_async_copy(col_h.at[pl.ds(0, _CH)], ibufs[b2],
                                isems[b2]).wait()
          pltpu.make_async_copy(msg_h.at[pl.ds(0, _CH)], bufs[b2],
                                lsems[b2]).wait()
          pltpu.async_copy(bufs[b2], shared.at[ibufs[b2]], asems[b2],
                           add=True)
      return carry

    lax.fori_loop(0, (n_it + 8) // 4 + 1, body, 0)
    plsc.subcore_barrier()

    def obody(z, carry):
      pltpu.sync_copy(shared.at[pl.ds(s * stripe + z * _CH, _CH)], bf0)
      pltpu.sync_copy(bf0, out_h.at[pl.ds(c * n_pad + s * stripe + z * _CH,
                                          _CH)])
      return carry

    lax.fori_loop(0, n_z, obody, 0)

  return k(msg, col, zeros)


_BE = 8000


def _stats_rows(v):
  s = jnp.sum(v, axis=0, keepdims=True)
  ss = jnp.sum(v * v, axis=0, keepdims=True)
  return jnp.concatenate(
      [s, ss, jnp.zeros((6, v.shape[1]), jnp.float32)], axis=0)


def _tc_atom0(x, atom_emb):
  n = x.shape[0]
  a, f = atom_emb.shape

  def body(x_ref, emb_ref, out_ref, st_ref):
    onehot = (x_ref[...][:, None]
              == lax.broadcasted_iota(jnp.int32, (1, a), 1)).astype(jnp.float32)
    v = jnp.dot(onehot, emb_ref[...], preferred_element_type=jnp.float32,
                precision=jax.lax.Precision.HIGHEST)
    out_ref[...] = v
    st_ref[...] = _stats_rows(v)

  return pl.pallas_call(
      body,
      out_shape=(jax.ShapeDtypeStruct((n, f), jnp.float32),
                 jax.ShapeDtypeStruct((8, f), jnp.float32)),
  )(x, atom_emb)


def _tc_bond0(attr3, bond_emb, e):
  b, f = bond_emb.shape
  grid = e // _BE

  def body(attr_ref, emb_ref, out_ref, st_ref):
    onehot = (attr_ref[0, 0, :][:, None]
              == lax.broadcasted_iota(jnp.int32, (1, b), 1)).astype(jnp.float32)
    v = jnp.dot(onehot, emb_ref[...], preferred_element_type=jnp.float32,
                precision=jax.lax.Precision.HIGHEST)
    out_ref[...] = v
    i = pl.program_id(0)

    @pl.when(i == 0)
    def _():
      st_ref[...] = jnp.zeros_like(st_ref)

    st_ref[...] += _stats_rows(v)

  return pl.pallas_call(
      body,
      grid=(grid,),
      in_specs=[pl.BlockSpec((1, 1, _BE), lambda i: (i, 0, 0)),
                pl.BlockSpec((b, f), lambda i: (0, 0))],
      out_specs=(pl.BlockSpec((_BE, f), lambda i: (i, 0)),
                 pl.BlockSpec((8, f), lambda i: (0, 0))),
      out_shape=(jax.ShapeDtypeStruct((e, f), jnp.float32),
                 jax.ShapeDtypeStruct((8, f), jnp.float32)),
  )(attr3, bond_emb)


def _tc_edge(src, tgt, bond, a_sc, a_sh, b_sc, b_sh,
             w1a, w1b, w1c, b1, w2, b2, mw, mb):
  e, f = bond.shape
  f2 = w1a.shape[1]
  grid = e // _BE

  def body(src_ref, tgt_ref, bond_ref, asc, ash, bsc, bsh,
           w1a_r, w1b_r, w1c_r, b1_r, w2_r, b2_r, mw_r, mb_r,
           bond_out, msg_out, st_ref):
    xs = src_ref[...] * asc[...] + ash[...]
    xt = tgt_ref[...] * asc[...] + ash[...]
    en = bond_ref[...] * bsc[...] + bsh[...]
    hcat = jnp.concatenate([xs, xt, en], axis=1)
    w1cat = jnp.concatenate([w1a_r[...], w1b_r[...], w1c_r[...]], axis=0)
    h = jnp.dot(hcat, w1cat, preferred_element_type=jnp.float32,
                precision=None) + b1_r[...]
    ne = jnp.dot(jnp.maximum(h, 0.0), w2_r[...],
                 preferred_element_type=jnp.float32, precision=None) + b2_r[...]
    bnew = bond_ref[...] + ne
    bond_out[...] = bnew
    msg_out[...] = (jnp.dot(xs, mw_r[...], preferred_element_type=jnp.float32, precision=None)
                    + mb_r[...]) * bnew
    i = pl.program_id(0)

    @pl.when(i == 0)
    def _():
      st_ref[...] = jnp.zeros_like(st_ref)

    st_ref[...] += _stats_rows(bnew)

  cst = lambda s: pl.BlockSpec(s, lambda i: tuple(0 for _ in s))
  blk = pl.BlockSpec((_BE, f), lambda i: (i, 0))
  return pl.pallas_call(
      body,
      grid=(grid,),
      in_specs=[blk, blk, blk,
                cst((1, f)), cst((1, f)), cst((1, f)), cst((1, f)),
                cst((f, f2)), cst((f, f2)), cst((f, f2)), cst((1, f2)),
                cst((f2, f)), cst((1, f)), cst((f, f)), cst((1, f))],
      out_specs=(blk, blk, pl.BlockSpec((8, f), lambda i: (0, 0))),
      out_shape=(jax.ShapeDtypeStruct((e, f), jnp.float32),
                 jax.ShapeDtypeStruct((e, f), jnp.float32),
                 jax.ShapeDtypeStruct((8, f), jnp.float32)),
  )(src, tgt, bond, a_sc, a_sh, b_sc, b_sh, w1a, w1b, w1c, b1, w2, b2, mw, mb)


def _tc_node(aggr2, atom, nw1, nb1, nw2, nb2, n_pad):
  n, f = atom.shape

  def body(a2_ref, atom_ref, w1_r, b1_r, w2_r, b2_r, out_ref, st_ref):
    a = a2_ref[0:n, :] + a2_ref[n_pad:n_pad + n, :]
    h = jnp.maximum(
        jnp.dot(a, w1_r[...], preferred_element_type=jnp.float32, precision=None) + b1_r[...],
        0.0)
    upd = jnp.dot(h, w2_r[...], preferred_element_type=jnp.float32, precision=None) + b2_r[...]
    anew = atom_ref[...] + upd
    out_ref[...] = anew
    st_ref[...] = _stats_rows(anew)

  return pl.pallas_call(
      body,
      out_shape=(jax.ShapeDtypeStruct((n, f), jnp.float32),
                 jax.ShapeDtypeStruct((8, f), jnp.float32)),
  )(aggr2, atom, nw1, nb1, nw2, nb2)


def _tc_final(bond, attr3, out_w, out_b, mean_emb):
  e, f = bond.shape
  b = mean_emb.shape[0]
  grid = e // _BE

  def body(bond_ref, attr_ref, w_r, b_r, memb_r, out_ref):
    onehot = (attr_ref[0, 0, :][:, None]
              == lax.broadcasted_iota(jnp.int32, (1, b), 1)).astype(jnp.float32)
    mean = jnp.dot(onehot, memb_r[...], preferred_element_type=jnp.float32,
                   precision=jax.lax.Precision.HIGHEST)
    out_ref[...] = (jnp.dot(bond_ref[...], w_r[...],
                            preferred_element_type=jnp.float32, precision=None)
                    + b_r[...] + mean)

  cst = lambda s: pl.BlockSpec(s, lambda i: tuple(0 for _ in s))
  return pl.pallas_call(
      body,
      grid=(grid,),
      in_specs=[pl.BlockSpec((_BE, f), lambda i: (i, 0)),
                pl.BlockSpec((1, 1, _BE), lambda i: (i, 0, 0)),
                cst((f, 1)), cst((1, 1)), cst((b, 1))],
      out_specs=pl.BlockSpec((_BE, 1), lambda i: (i, 0)),
      out_shape=jax.ShapeDtypeStruct((e, 1), jnp.float32),
  )(bond, attr3, out_w, out_b, mean_emb)


def _bn_coeffs(st, cnt, gamma, beta):
  m = st[0] / cnt
  var = st[1] / cnt - m * m
  scale = gamma / jnp.sqrt(var + _EPS)
  shift = beta - m * scale
  return scale.reshape(1, -1), shift.reshape(1, -1)


def kernel(x, edge_index, edge_attr, atom_emb, bond_emb, bond_mean_emb,
           bn_atom_gamma, bn_atom_beta, bn_bond_gamma, bn_bond_beta,
           edge_W1, edge_b1, edge_W2, edge_b2, msg_W, msg_b,
           node_W1, node_b1, node_W2, node_b2, out_W, out_b):
  n = x.shape[0]
  e = edge_attr.shape[0]
  f = atom_emb.shape[1]
  num_layers = edge_W1.shape[0]
  row = edge_index[0].astype(jnp.int32)
  col = edge_index[1].astype(jnp.int32)
  x = x.astype(jnp.int32)
  attr = edge_attr.astype(jnp.int32)
  attr3 = attr.reshape(e // _BE, 1, _BE)
  stripe = -(-(n // _NS) // _CH) * _CH
  n_pad = _NS * stripe
  zeros = jnp.zeros((_CH, f), jnp.float32)

  atom_state, ast = _tc_atom0(x, atom_emb)
  bond_state, bst = _tc_bond0(attr3, bond_emb, e)

  for l in range(num_layers):
    a_sc, a_sh = _bn_coeffs(ast, n, bn_atom_gamma[l], bn_atom_beta[l])
    b_sc, b_sh = _bn_coeffs(bst, e, bn_bond_gamma[l], bn_bond_beta[l])
    src, tgt = _sc_gather_src_tgt(atom_state, row, col)
    w1 = edge_W1[l]
    bond_state, msg, bst = _tc_edge(
        src, tgt, bond_state, a_sc, a_sh, b_sc, b_sh,
        w1[:f], w1[f:2 * f], w1[2 * f:], edge_b1[l].reshape(1, -1),
        edge_W2[l], edge_b2[l].reshape(1, -1),
        msg_W[l], msg_b[l].reshape(1, -1))
    aggr2 = _sc_scatter_add(msg, col, zeros, n_pad)
    atom_state, ast = _tc_node(
        aggr2, atom_state, node_W1[l], node_b1[l].reshape(1, -1),
        node_W2[l], node_b2[l].reshape(1, -1), n_pad)

  pred = _tc_final(bond_state, attr3, out_W, out_b.reshape(1, 1),
                   bond_mean_emb)
  return pred.reshape(e)
